# initial kernel scaffold (unmeasured)
import functools

import jax
import jax.numpy as jnp
from jax import lax
from jax.experimental import pallas as pl
from jax.experimental.pallas import tpu as pltpu

N_DEV = 4
BM = 512


def kernel(x, dy, gamma):
    m_per, d = x.shape
    n_steps = m_per // BM

    def body(x_ref, dy_ref, gamma_ref, out_ref, acc_ref, comm_ref,
             send_sems, recv_sems):
        step = pl.program_id(0)

        xb = x_ref[:, :]
        dyb = dy_ref[:, :]
        mu = jnp.mean(xb, axis=1, keepdims=True)
        xc = xb - mu
        var = jnp.mean(xc * xc, axis=1, keepdims=True)
        xhat = xc * lax.rsqrt(var + 1e-5)
        pg = jnp.sum(dyb * xhat, axis=0, keepdims=True)
        pb = jnp.sum(dyb, axis=0, keepdims=True)
        part = jnp.concatenate([pg, pb], axis=0)

        @pl.when(step == 0)
        def _():
            acc_ref[:, :] = part

        @pl.when(step != 0)
        def _():
            acc_ref[:, :] = acc_ref[:, :] + part

        @pl.when(step == n_steps - 1)
        def _():
            my = lax.axis_index("i")

            barrier = pltpu.get_barrier_semaphore()
            for off in range(1, N_DEV):
                pl.semaphore_signal(
                    barrier, inc=1,
                    device_id=((my + off) % N_DEV,),
                    device_id_type=pl.DeviceIdType.MESH,
                )
            pl.semaphore_wait(barrier, N_DEV - 1)

            rdmas = []
            for off in range(1, N_DEV):
                rdma = pltpu.make_async_remote_copy(
                    src_ref=acc_ref,
                    dst_ref=comm_ref.at[off - 1],
                    send_sem=send_sems.at[off - 1],
                    recv_sem=recv_sems.at[off - 1],
                    device_id=((my + off) % N_DEV,),
                    device_id_type=pl.DeviceIdType.MESH,
                )
                rdma.start()
                rdmas.append(rdma)
            for rdma in rdmas:
                rdma.wait()

            out_ref[:, :] = (acc_ref[:, :] + comm_ref[0, :, :]
                             + comm_ref[1, :, :] + comm_ref[2, :, :])

    return pl.pallas_call(
        body,
        grid=(n_steps,),
        out_shape=jax.ShapeDtypeStruct((2, d), jnp.float32),
        in_specs=[
            pl.BlockSpec((BM, d), lambda i: (i, 0)),
            pl.BlockSpec((BM, d), lambda i: (i, 0)),
            pl.BlockSpec(memory_space=pltpu.ANY),
        ],
        out_specs=pl.BlockSpec((2, d), lambda i: (0, 0)),
        scratch_shapes=[
            pltpu.VMEM((2, d), jnp.float32),
            pltpu.VMEM((N_DEV - 1, 2, d), jnp.float32),
            pltpu.SemaphoreType.DMA((N_DEV - 1,)),
            pltpu.SemaphoreType.DMA((N_DEV - 1,)),
        ],
        compiler_params=pltpu.CompilerParams(collective_id=0),
    )(x, dy, gamma)


# baseline (device time: 28703 ns/iter reference)
import functools

import jax
import jax.numpy as jnp
from jax import lax
from jax.experimental import pallas as pl
from jax.experimental.pallas import tpu as pltpu

N_DEV = 4
BM = 512


def kernel(x, dy, gamma):
    m_per, d = x.shape
    n_steps = m_per // BM

    def body(x_ref, dy_ref, gamma_ref, out_ref, acc_ref, comm_ref,
             send_sems, recv_sems):
        step = pl.program_id(0)

        xb = x_ref[:, :]
        dyb = dy_ref[:, :]
        mu = jnp.mean(xb, axis=1, keepdims=True)
        xc = xb - mu
        var = jnp.mean(xc * xc, axis=1, keepdims=True)
        xhat = xc * lax.rsqrt(var + 1e-5)
        pg = jnp.sum(dyb * xhat, axis=0, keepdims=True)
        pb = jnp.sum(dyb, axis=0, keepdims=True)
        part = jnp.concatenate([pg, pb], axis=0)

        @pl.when(step == 0)
        def _():
            acc_ref[:, :] = part

        @pl.when(step != 0)
        def _():
            acc_ref[:, :] = acc_ref[:, :] + part

        @pl.when(step == n_steps - 1)
        def _():
            my = lax.axis_index("i")

            barrier = pltpu.get_barrier_semaphore()
            for off in range(1, N_DEV):
                pl.semaphore_signal(
                    barrier, inc=1,
                    device_id=((my + off) % N_DEV,),
                    device_id_type=pl.DeviceIdType.MESH,
                )
            pl.semaphore_wait(barrier, N_DEV - 1)

            rdmas = []
            for off in range(1, N_DEV):
                rdma = pltpu.make_async_remote_copy(
                    src_ref=acc_ref,
                    dst_ref=comm_ref.at[off - 1],
                    send_sem=send_sems.at[off - 1],
                    recv_sem=recv_sems.at[off - 1],
                    device_id=((my + off) % N_DEV,),
                    device_id_type=pl.DeviceIdType.MESH,
                )
                rdma.start()
                rdmas.append(rdma)
            for rdma in rdmas:
                rdma.wait()

            out_ref[:, :] = (acc_ref[:, :] + comm_ref[0, :, :]
                             + comm_ref[1, :, :] + comm_ref[2, :, :])

    return pl.pallas_call(
        body,
        grid=(n_steps,),
        out_shape=jax.ShapeDtypeStruct((2, d), jnp.float32),
        in_specs=[
            pl.BlockSpec((BM, d), lambda i: (i, 0)),
            pl.BlockSpec((BM, d), lambda i: (i, 0)),
            pl.BlockSpec(memory_space=pl.ANY),
        ],
        out_specs=pl.BlockSpec((2, d), lambda i: (0, 0)),
        scratch_shapes=[
            pltpu.VMEM((2, d), jnp.float32),
            pltpu.VMEM((N_DEV - 1, 2, d), jnp.float32),
            pltpu.SemaphoreType.DMA((N_DEV - 1,)),
            pltpu.SemaphoreType.DMA((N_DEV - 1,)),
        ],
        compiler_params=pltpu.CompilerParams(collective_id=0),
    )(x, dy, gamma)
